# baseline (speedup 1.0000x reference)
import functools

import jax
import jax.numpy as jnp
from jax import lax
from jax.experimental import pallas as pl
from jax.experimental.pallas import tpu as pltpu
from jax.experimental.pallas import tpu_sc as plsc

_NEU_IN = 100000
_NEU_OUT = 128
_THRES = 1.0
_DECAY = 2.0 ** 4

_NW = 32
_ROWS_W = 3200
_PAD_IN = _NW * _ROWS_W
_CHUNKS = _ROWS_W // 16
_G = 64
_NB = 8
_OCT = _NB * _G
_IDX_CAP = 4224
_V8 = _NEU_OUT // 16


def _sc_body(spikes_hbm, w_hbm, out_hbm, spk_v, idx_v,
             b0, b1, b2, b3, b4, b5, b6, b7, w0_v, acc_v,
             s0, s1, s2, s3, s4, s5, s6, s7):
    bufs = (b0, b1, b2, b3, b4, b5, b6, b7)
    sems = (s0, s1, s2, s3, s4, s5, s6, s7)

    wid = lax.axis_index("s") * 2 + lax.axis_index("c")
    base = wid * _ROWS_W

    pltpu.sync_copy(spikes_hbm.at[pl.ds(base, _ROWS_W)], spk_v)
    pltpu.sync_copy(w_hbm.at[pl.ds(0, 1)], w0_v)

    lanes = lax.iota(jnp.int32, 16)
    zero16 = jnp.zeros((16,), jnp.int32)

    def zfill(c, x):
        idx_v[pl.ds(c * 16, 16)] = zero16
        return x

    lax.fori_loop(0, _IDX_CAP // 16, zfill, 0)

    def build(c, cnt):
        sv = spk_v[pl.ds(c * 16, 16)]
        m = sv > 0
        incl = plsc.cumsum(jnp.where(m, jnp.ones((16,), jnp.int32), zero16))
        dest = cnt + incl - 1
        rowidx = (base + c * 16) + lanes
        plsc.store_scatter(idx_v, [dest], rowidx, mask=m)
        return cnt + plsc.all_reduce_population_count(m)

    cnt_v = lax.fori_loop(0, _CHUNKS, build, jnp.zeros((16,), jnp.int32))
    cnt = cnt_v[0]
    nring = lax.div(cnt + (_OCT - 1), _OCT)

    def fire(off, buf, sem):
        pltpu.async_copy(w_hbm.at[idx_v.at[pl.ds(off, _G)]], buf, sem)

    def drain(buf, sem):
        pltpu.make_async_copy(w_hbm.at[idx_v.at[pl.ds(0, _G)]], buf, sem).wait()

    for b in range(_NB):
        fire(b * _G, bufs[b], sems[b])

    def accum(buf, acc):
        def body4(q, a):
            new = list(a)
            for jj in range(4):
                for v in range(_V8):
                    new[v] = new[v] + buf[q * 4 + jj, pl.ds(v * 16, 16)]
            return tuple(new)
        return lax.fori_loop(0, _G // 4, body4, acc)

    init = tuple(jnp.zeros((16,), jnp.float32) for _ in range(_V8))

    def rev(i, accs):
        off = i * _OCT
        for b in range(_NB):
            drain(bufs[b], sems[b])
            accs = accum(bufs[b], accs)
            fire(off + (b + _NB) * _G, bufs[b], sems[b])
        return accs

    accs = lax.fori_loop(0, nring, rev, init)

    for b in range(_NB):
        drain(bufs[b], sems[b])

    npad_v = (nring * _OCT - cnt_v).astype(jnp.float32)
    for v in range(_V8):
        acc_v[0, pl.ds(v * 16, 16)] = (
            accs[v] - npad_v * w0_v[0, pl.ds(v * 16, 16)]
        )
    pltpu.sync_copy(acc_v, out_hbm.at[pl.ds(wid, 1)])


_sc_call = functools.partial(
    pl.kernel,
    out_type=jax.ShapeDtypeStruct((_NW, _NEU_OUT), jnp.float32),
    mesh=plsc.VectorSubcoreMesh(core_axis_name="c", subcore_axis_name="s"),
    compiler_params=pltpu.CompilerParams(needs_layout_passes=False),
    scratch_types=[
        pltpu.VMEM((_ROWS_W,), jnp.int32),
        pltpu.VMEM((_IDX_CAP,), jnp.int32),
    ] + [pltpu.VMEM((_G, _NEU_OUT), jnp.float32) for _ in range(_NB)] + [
        pltpu.VMEM((1, _NEU_OUT), jnp.float32),
        pltpu.VMEM((1, _NEU_OUT), jnp.float32),
    ] + [pltpu.SemaphoreType.DMA for _ in range(_NB)],
)(_sc_body)


def _ep_body(part_ref, mp_ref, spk_ref, mnew_ref):
    contrib = jnp.sum(part_ref[...], axis=0, keepdims=True)
    m = mp_ref[...] + contrib
    s = m >= _THRES
    mnew = jnp.where(s, m - _THRES, (m * _DECAY - m) / _DECAY)
    spk_ref[...] = s.astype(jnp.float32)
    mnew_ref[...] = mnew


_ep_call = pl.pallas_call(
    _ep_body,
    out_shape=(
        jax.ShapeDtypeStruct((1, _NEU_OUT), jnp.float32),
        jax.ShapeDtypeStruct((1, _NEU_OUT), jnp.float32),
    ),
)


def kernel(spikes_in, W, mempot):
    spikes_pad = (
        jnp.zeros((_PAD_IN,), jnp.int32).at[:_NEU_IN].set(spikes_in.astype(jnp.int32))
    )
    partials = _sc_call(spikes_pad, W)
    spk_f, mnew = _ep_call(partials, mempot.reshape(1, _NEU_OUT))
    spikes_out = spk_f[0] > 0.5
    traces_out = jnp.zeros((_NEU_OUT,), jnp.float32)
    return (spikes_out, traces_out, mnew[0])


# speedup vs baseline: 24.5449x; 24.5449x over previous
import functools

import jax
import jax.numpy as jnp
from jax import lax
from jax.experimental import pallas as pl
from jax.experimental.pallas import tpu as pltpu
from jax.experimental.pallas import tpu_sc as plsc

_NEU_IN = 100000
_NEU_OUT = 128
_THRES = 1.0
_DECAY = 2.0 ** 4

_NW = 32
_ROWS_W = 3200
_PAD_IN = _NW * _ROWS_W
_CHUNKS = _ROWS_W // 16
_GR = 160
_NB = 5
_V8 = _NEU_OUT // 16


def _sc_body(spikes_hbm, w_hbm, out_hbm, spk_v, spkf_v,
             b0, b1, b2, b3, b4, acc_v, s0, s1, s2, s3, s4):
    bufs = (b0, b1, b2, b3, b4)
    sems = (s0, s1, s2, s3, s4)

    wid = lax.axis_index("s") * 2 + lax.axis_index("c")
    base = wid * _ROWS_W
    nvalid = jnp.minimum(_ROWS_W, _NEU_IN - base)
    trips = lax.div(nvalid, _GR)
    revs = lax.div(trips, _NB)

    pltpu.sync_copy(spikes_hbm.at[pl.ds(base, _ROWS_W)], spk_v)

    def conv(c, x):
        spkf_v[pl.ds(c * 16, 16)] = spk_v[pl.ds(c * 16, 16)].astype(jnp.float32)
        return x

    lax.fori_loop(0, _CHUNKS, conv, 0)

    def fire(c, buf, sem):
        pltpu.async_copy(w_hbm.at[pl.ds(base + c * _GR, _GR)], buf, sem)

    def drain(buf, sem):
        pltpu.make_async_copy(w_hbm.at[pl.ds(0, _GR)], buf, sem).wait()

    for b in range(_NB):
        fire(b, bufs[b], sems[b])

    def accum(buf, c, acc):
        rbase = c * _GR

        def body4(q, a):
            new = list(a)
            for jj in range(4):
                ridx = jnp.full((16,), rbase + q * 4 + jj, jnp.int32)
                s = plsc.load_gather(spkf_v, [ridx])
                for v in range(_V8):
                    new[v] = new[v] + buf[q * 4 + jj, pl.ds(v * 16, 16)] * s
            return tuple(new)

        return lax.fori_loop(0, _GR // 4, body4, acc)

    init = tuple(jnp.zeros((16,), jnp.float32) for _ in range(_V8))

    def rev(i, accs):
        for b in range(_NB):
            c = i * _NB + b
            drain(bufs[b], sems[b])
            accs = accum(bufs[b], c, accs)
            cf = jnp.minimum(c + _NB, trips - 1)
            fire(cf, bufs[b], sems[b])
        return accs

    accs = lax.fori_loop(0, revs, rev, init)

    for b in range(_NB):
        drain(bufs[b], sems[b])

    for v in range(_V8):
        acc_v[0, pl.ds(v * 16, 16)] = accs[v]
    pltpu.sync_copy(acc_v, out_hbm.at[pl.ds(wid, 1)])


_sc_call = functools.partial(
    pl.kernel,
    out_type=jax.ShapeDtypeStruct((_NW, _NEU_OUT), jnp.float32),
    mesh=plsc.VectorSubcoreMesh(core_axis_name="c", subcore_axis_name="s"),
    compiler_params=pltpu.CompilerParams(needs_layout_passes=False),
    scratch_types=[
        pltpu.VMEM((_ROWS_W,), jnp.int32),
        pltpu.VMEM((_ROWS_W,), jnp.float32),
    ] + [pltpu.VMEM((_GR, _NEU_OUT), jnp.float32) for _ in range(_NB)] + [
        pltpu.VMEM((1, _NEU_OUT), jnp.float32),
    ] + [pltpu.SemaphoreType.DMA for _ in range(_NB)],
)(_sc_body)


def _ep_body(part_ref, mp_ref, spk_ref, mnew_ref):
    contrib = jnp.sum(part_ref[...], axis=0, keepdims=True)
    m = mp_ref[...] + contrib
    s = m >= _THRES
    mnew = jnp.where(s, m - _THRES, (m * _DECAY - m) / _DECAY)
    spk_ref[...] = s.astype(jnp.float32)
    mnew_ref[...] = mnew


_ep_call = pl.pallas_call(
    _ep_body,
    out_shape=(
        jax.ShapeDtypeStruct((1, _NEU_OUT), jnp.float32),
        jax.ShapeDtypeStruct((1, _NEU_OUT), jnp.float32),
    ),
)


def kernel(spikes_in, W, mempot):
    spikes_pad = (
        jnp.zeros((_PAD_IN,), jnp.int32).at[:_NEU_IN].set(spikes_in.astype(jnp.int32))
    )
    partials = _sc_call(spikes_pad, W)
    spk_f, mnew = _ep_call(partials, mempot.reshape(1, _NEU_OUT))
    spikes_out = spk_f[0] > 0.5
    traces_out = jnp.zeros((_NEU_OUT,), jnp.float32)
    return (spikes_out, traces_out, mnew[0])


# speedup vs baseline: 25.2126x; 1.0272x over previous
import functools

import jax
import jax.numpy as jnp
from jax import lax
from jax.experimental import pallas as pl
from jax.experimental.pallas import tpu as pltpu
from jax.experimental.pallas import tpu_sc as plsc

_NEU_IN = 100000
_NEU_OUT = 128
_THRES = 1.0
_DECAY = 2.0 ** 4

_NW = 32
_ROWS_W = 3200
_PAD_IN = _NW * _ROWS_W
_CHUNKS = _ROWS_W // 16
_GR = 160
_NB = 5
_V8 = _NEU_OUT // 16


def _sc_body(spikes_hbm, w_hbm, out_hbm, spk_v, spkf_v,
             b0, b1, b2, b3, b4, acc_v, s0, s1, s2, s3, s4):
    bufs = (b0, b1, b2, b3, b4)
    sems = (s0, s1, s2, s3, s4)

    wid = lax.axis_index("s") * 2 + lax.axis_index("c")
    base = wid * _ROWS_W
    nvalid = jnp.minimum(_ROWS_W, _NEU_IN - base)
    trips = lax.div(nvalid, _GR)
    revs = lax.div(trips, _NB)

    pltpu.sync_copy(spikes_hbm.at[pl.ds(base, _ROWS_W)], spk_v)

    def conv(c, x):
        spkf_v[pl.ds(c * 16, 16)] = spk_v[pl.ds(c * 16, 16)].astype(jnp.float32)
        return x

    lax.fori_loop(0, _CHUNKS, conv, 0)

    def fire(c, buf, sem):
        pltpu.async_copy(w_hbm.at[pl.ds(base + c * _GR, _GR)], buf, sem)

    def drain(buf, sem):
        pltpu.make_async_copy(w_hbm.at[pl.ds(0, _GR)], buf, sem).wait()

    for b in range(_NB):
        fire(b, bufs[b], sems[b])

    def accum(buf, c, acc):
        rbase = c * _GR

        def body4(q, a):
            new = list(a)
            for jj in range(4):
                ridx = jnp.full((16,), rbase + q * 4 + jj, jnp.int32)
                s = plsc.load_gather(spkf_v, [ridx])
                for v in range(_V8):
                    new[v] = new[v] + buf[q * 4 + jj, pl.ds(v * 16, 16)] * s
            return tuple(new)

        return lax.fori_loop(0, _GR // 4, body4, acc)

    init = tuple(jnp.zeros((16,), jnp.float32) for _ in range(_V8))

    def rev(i, accs):
        for b in range(_NB):
            c = i * _NB + b
            drain(bufs[b], sems[b])
            accs = accum(bufs[b], c, accs)
            cf = jnp.minimum(c + _NB, trips - 1)
            fire(cf, bufs[b], sems[b])
        return accs

    accs = lax.fori_loop(0, revs, rev, init)

    for b in range(_NB):
        drain(bufs[b], sems[b])

    for v in range(_V8):
        acc_v[0, pl.ds(v * 16, 16)] = accs[v]
    pltpu.sync_copy(acc_v, out_hbm.at[pl.ds(wid, 1)])


_sc_call = functools.partial(
    pl.kernel,
    out_type=jax.ShapeDtypeStruct((_NW, _NEU_OUT), jnp.float32),
    mesh=plsc.VectorSubcoreMesh(core_axis_name="c", subcore_axis_name="s"),
    compiler_params=pltpu.CompilerParams(needs_layout_passes=False),
    scratch_types=[
        pltpu.VMEM((_ROWS_W,), jnp.int32),
        pltpu.VMEM((_ROWS_W,), jnp.float32),
    ] + [pltpu.VMEM((_GR, _NEU_OUT), jnp.float32) for _ in range(_NB)] + [
        pltpu.VMEM((1, _NEU_OUT), jnp.float32),
    ] + [pltpu.SemaphoreType.DMA for _ in range(_NB)],
)(_sc_body)


def kernel(spikes_in, W, mempot):
    spikes_pad = (
        jnp.zeros((_PAD_IN,), jnp.int32).at[:_NEU_IN].set(spikes_in.astype(jnp.int32))
    )
    partials = _sc_call(spikes_pad, W)
    m = mempot + jnp.sum(partials, axis=0)
    spikes_out = m >= _THRES
    mnew = jnp.where(spikes_out, m - _THRES, (m * _DECAY - m) / _DECAY)
    traces_out = jnp.zeros((_NEU_OUT,), jnp.float32)
    return (spikes_out, traces_out, mnew)
